# baseline (speedup 1.0000x reference)
import functools

import jax
import jax.numpy as jnp
from jax import lax
from jax.experimental import pallas as pl
from jax.experimental.pallas import tpu as pltpu
from jax.experimental.pallas import tpu_sc as plsc

N_NODES = 50000
N_EDGES = 800000
SEQ_LEN = 20
IN_FEAT = 5
HIDDEN = 128
GCN_HID = 16

NPAD = 50048
ROWS_PER_SUB = NPAD // 16
CH = 125
NCHUNK = N_EDGES // CH
NW = 32
CPW = NCHUNK // NW
KD = 8
NGROUP = CPW // KD


def _lstm_body(x_ref, wih_ref, whh_ref, b_ref, w1_ref, out_ref):
    x = x_ref[...]
    nb = x.shape[0]
    h = jnp.zeros((nb, HIDDEN), jnp.float32)
    c = jnp.zeros((nb, HIDDEN), jnp.float32)
    bf = jnp.bfloat16
    wih = wih_ref[...].astype(bf)
    whh = whh_ref[...].astype(bf)
    b = b_ref[...]
    dn = (((1,), (0,)), ((), ()))
    for t in range(SEQ_LEN):
        x_t = x[:, IN_FEAT * t:IN_FEAT * (t + 1)]
        gates = lax.dot_general(x_t.astype(bf), wih, dn,
                                preferred_element_type=jnp.float32)
        gates = gates + lax.dot_general(h.astype(bf), whh, dn,
                                        preferred_element_type=jnp.float32)
        gates = gates + b
        i = jax.nn.sigmoid(gates[:, 0:HIDDEN])
        f = jax.nn.sigmoid(gates[:, HIDDEN:2 * HIDDEN])
        g = jnp.tanh(gates[:, 2 * HIDDEN:3 * HIDDEN])
        o = jax.nn.sigmoid(gates[:, 3 * HIDDEN:4 * HIDDEN])
        c = f * c + i * g
        h = o * jnp.tanh(c)
    out_ref[...] = lax.dot_general(h, w1_ref[...], dn,
                                   preferred_element_type=jnp.float32)


def _lstm_xw1(x2d, wih_t, whh_t, bias, w1):
    nb = 3128
    grid = NPAD // nb
    return pl.pallas_call(
        _lstm_body,
        grid=(grid,),
        in_specs=[
            pl.BlockSpec((nb, SEQ_LEN * IN_FEAT), lambda i: (i, 0)),
            pl.BlockSpec((IN_FEAT, 4 * HIDDEN), lambda i: (0, 0)),
            pl.BlockSpec((HIDDEN, 4 * HIDDEN), lambda i: (0, 0)),
            pl.BlockSpec((1, 4 * HIDDEN), lambda i: (0, 0)),
            pl.BlockSpec((HIDDEN, GCN_HID), lambda i: (0, 0)),
        ],
        out_specs=pl.BlockSpec((nb, GCN_HID), lambda i: (i, 0)),
        out_shape=jax.ShapeDtypeStruct((NPAD, GCN_HID), jnp.float32),
    )(x2d, wih_t, whh_t, bias, w1)


def _sc_mesh():
    return plsc.VectorSubcoreMesh(core_axis_name="c", subcore_axis_name="s",
                                  num_cores=2, num_subcores=16)


@functools.partial(
    pl.kernel,
    out_type=jax.ShapeDtypeStruct((2, NPAD, 1), jnp.float32),
    mesh=_sc_mesh(),
    scratch_types=[
        pltpu.VMEM((CPW, CH), jnp.int32),
        pltpu.VMEM((CH, 1), jnp.float32),
        pltpu.VMEM_SHARED((NPAD, 1), jnp.float32),
        pltpu.SemaphoreType.DMA,
        pltpu.SemaphoreType.DMA,
    ],
    compiler_params=pltpu.CompilerParams(use_tc_tiling_on_sc=False),
)
def _sc_degree(dst_hbm, ones_hbm, zeros_hbm, out_hbm, didx, onev, acc,
               isem, ssem):
    cid = lax.axis_index("c")
    sid = lax.axis_index("s")
    wid = sid * 2 + cid
    base_rows = sid * ROWS_PER_SUB
    pltpu.async_copy(dst_hbm.at[pl.ds(wid * CPW, CPW)], didx, isem)
    pltpu.sync_copy(ones_hbm, onev)
    pltpu.sync_copy(zeros_hbm, acc.at[pl.ds(base_rows, ROWS_PER_SUB)])
    pltpu.make_async_copy(dst_hbm.at[pl.ds(wid * CPW, CPW)], didx, isem).wait()
    plsc.subcore_barrier()

    def group(gi, _):
        for b in range(KD):
            k = gi * KD + b
            pltpu.async_copy(onev, acc.at[didx.at[k]], ssem, add=True)
        for b in range(KD):
            k = gi * KD + b
            pltpu.make_async_copy(onev, acc.at[didx.at[k]], ssem).wait()
        return 0

    lax.fori_loop(0, NGROUP, group, 0)
    plsc.subcore_barrier()
    pltpu.sync_copy(acc.at[pl.ds(base_rows, ROWS_PER_SUB)],
                    out_hbm.at[cid, pl.ds(base_rows, ROWS_PER_SUB)])


@functools.partial(
    pl.kernel,
    out_type=jax.ShapeDtypeStruct((2, NPAD, GCN_HID), jnp.float32),
    mesh=_sc_mesh(),
    scratch_types=[
        pltpu.VMEM((CPW, CH), jnp.int32),
        pltpu.VMEM((CPW, CH), jnp.int32),
        pltpu.VMEM((KD, CH, GCN_HID), jnp.float32),
        pltpu.VMEM_SHARED((NPAD, GCN_HID), jnp.float32),
        pltpu.SemaphoreType.DMA,
        pltpu.SemaphoreType.DMA,
        pltpu.SemaphoreType.DMA,
    ],
    compiler_params=pltpu.CompilerParams(use_tc_tiling_on_sc=False),
)
def _sc_propagate(g_hbm, src_hbm, dst_hbm, zeros_hbm, out_hbm,
                  sidx, didx, rows, acc, isem, gsem, ssem):
    cid = lax.axis_index("c")
    sid = lax.axis_index("s")
    wid = sid * 2 + cid
    base_rows = sid * ROWS_PER_SUB
    pltpu.async_copy(src_hbm.at[pl.ds(wid * CPW, CPW)], sidx, isem)
    pltpu.async_copy(dst_hbm.at[pl.ds(wid * CPW, CPW)], didx, isem)
    pltpu.sync_copy(zeros_hbm, acc.at[pl.ds(base_rows, ROWS_PER_SUB)])
    pltpu.make_async_copy(src_hbm.at[pl.ds(wid * CPW, CPW)], sidx, isem).wait()
    pltpu.make_async_copy(dst_hbm.at[pl.ds(wid * CPW, CPW)], didx, isem).wait()
    plsc.subcore_barrier()

    def group(gi, _):
        for b in range(KD):
            k = gi * KD + b
            pltpu.async_copy(g_hbm.at[sidx.at[k]], rows.at[b], gsem)
        for b in range(KD):
            k = gi * KD + b
            pltpu.make_async_copy(g_hbm.at[sidx.at[k]], rows.at[b],
                                  gsem).wait()
            pltpu.async_copy(rows.at[b], acc.at[didx.at[k]], ssem, add=True)
        for b in range(KD):
            k = gi * KD + b
            pltpu.make_async_copy(rows.at[b], acc.at[didx.at[k]], ssem).wait()
        return 0

    lax.fori_loop(0, NGROUP, group, 0)
    plsc.subcore_barrier()
    pltpu.sync_copy(acc.at[pl.ds(base_rows, ROWS_PER_SUB)],
                    out_hbm.at[cid, pl.ds(base_rows, ROWS_PER_SUB)])


def _scale1_body(degp_ref, xw1_ref, dinv_ref, g1_ref):
    deg = degp_ref[0] + degp_ref[1] + 1.0
    dinv = lax.rsqrt(deg)
    dinv_ref[...] = dinv
    g1_ref[...] = dinv * xw1_ref[...]


def _scale2_body(accp_ref, xw1_ref, dinv_ref, b1_ref, y_ref, g2_ref):
    dinv = dinv_ref[...]
    acc = accp_ref[0] + accp_ref[1]
    p1 = dinv * acc + (dinv * dinv) * xw1_ref[...]
    y = jnp.maximum(p1 + b1_ref[...], 0.0)
    y_ref[...] = y
    g2_ref[...] = dinv * y


def _final_body(accp_ref, y_ref, dinv_ref, w2_ref, b2_ref, wfc_ref, bfc_ref,
                out_ref):
    dinv = dinv_ref[...]
    acc = accp_ref[0] + accp_ref[1]
    p2 = dinv * acc + (dinv * dinv) * y_ref[...]
    dn_t = (((1,), (1,)), ((), ()))
    v = lax.dot_general(w2_ref[...], wfc_ref[...], dn_t,
                        preferred_element_type=jnp.float32)
    c = lax.dot_general(b2_ref[...], wfc_ref[...], dn_t,
                        preferred_element_type=jnp.float32)
    dn = (((1,), (0,)), ((), ()))
    z = lax.dot_general(p2, v, dn, preferred_element_type=jnp.float32)
    z = z + c + bfc_ref[...]
    out_ref[...] = jnp.where(z > 0, z, 0.2 * z)


_NB2 = 3128
_GRID2 = NPAD // _NB2


def _scale1(degp, xw1):
    return pl.pallas_call(
        _scale1_body,
        grid=(_GRID2,),
        in_specs=[
            pl.BlockSpec((2, _NB2, 1), lambda i: (0, i, 0)),
            pl.BlockSpec((_NB2, GCN_HID), lambda i: (i, 0)),
        ],
        out_specs=[
            pl.BlockSpec((_NB2, 1), lambda i: (i, 0)),
            pl.BlockSpec((_NB2, GCN_HID), lambda i: (i, 0)),
        ],
        out_shape=[
            jax.ShapeDtypeStruct((NPAD, 1), jnp.float32),
            jax.ShapeDtypeStruct((NPAD, GCN_HID), jnp.float32),
        ],
    )(degp, xw1)


def _scale2(accp, xw1, dinv, b1):
    return pl.pallas_call(
        _scale2_body,
        grid=(_GRID2,),
        in_specs=[
            pl.BlockSpec((2, _NB2, GCN_HID), lambda i: (0, i, 0)),
            pl.BlockSpec((_NB2, GCN_HID), lambda i: (i, 0)),
            pl.BlockSpec((_NB2, 1), lambda i: (i, 0)),
            pl.BlockSpec((1, GCN_HID), lambda i: (0, 0)),
        ],
        out_specs=[
            pl.BlockSpec((_NB2, GCN_HID), lambda i: (i, 0)),
            pl.BlockSpec((_NB2, GCN_HID), lambda i: (i, 0)),
        ],
        out_shape=[
            jax.ShapeDtypeStruct((NPAD, GCN_HID), jnp.float32),
            jax.ShapeDtypeStruct((NPAD, GCN_HID), jnp.float32),
        ],
    )(accp, xw1, dinv, b1)


def _final(accp, y, dinv, w2, b2, wfc, bfc):
    return pl.pallas_call(
        _final_body,
        grid=(_GRID2,),
        in_specs=[
            pl.BlockSpec((2, _NB2, GCN_HID), lambda i: (0, i, 0)),
            pl.BlockSpec((_NB2, GCN_HID), lambda i: (i, 0)),
            pl.BlockSpec((_NB2, 1), lambda i: (i, 0)),
            pl.BlockSpec((GCN_HID, HIDDEN), lambda i: (0, 0)),
            pl.BlockSpec((1, HIDDEN), lambda i: (0, 0)),
            pl.BlockSpec((1, HIDDEN), lambda i: (0, 0)),
            pl.BlockSpec((1, 1), lambda i: (0, 0)),
        ],
        out_specs=pl.BlockSpec((_NB2, 1), lambda i: (i, 0)),
        out_shape=jax.ShapeDtypeStruct((NPAD, 1), jnp.float32),
    )(accp, y, dinv, w2, b2, wfc, bfc)


def kernel(inputs, edge_index, W_ih, W_hh, b_ih, b_hh, W1, b1, W2, b2,
           W_fc, b_fc):
    src = edge_index[0].astype(jnp.int32).reshape(NCHUNK, CH)
    dst = edge_index[1].astype(jnp.int32).reshape(NCHUNK, CH)
    x2d = inputs.reshape(N_NODES, SEQ_LEN * IN_FEAT)
    x2d = jnp.pad(x2d, ((0, NPAD - N_NODES), (0, 0)))
    wih_t = W_ih.T
    whh_t = W_hh.T
    bias = (b_ih + b_hh).reshape(1, 4 * HIDDEN)

    xw1 = _lstm_xw1(x2d, wih_t, whh_t, bias, W1)

    ones_e = jnp.ones((CH, 1), jnp.float32)
    zeros1 = jnp.zeros((ROWS_PER_SUB, 1), jnp.float32)
    zeros16 = jnp.zeros((ROWS_PER_SUB, GCN_HID), jnp.float32)

    degp = _sc_degree(dst, ones_e, zeros1)
    dinv, g1 = _scale1(degp, xw1)

    acc1 = _sc_propagate(g1, src, dst, zeros16)
    y, g2 = _scale2(acc1, xw1, dinv, b1.reshape(1, GCN_HID))

    acc2 = _sc_propagate(g2, src, dst, zeros16)
    pred = _final(acc2, y, dinv, W2,
                  b2.reshape(1, HIDDEN), W_fc, b_fc.reshape(1, 1))
    return pred[:N_NODES]


# speedup vs baseline: 1.1874x; 1.1874x over previous
import functools

import jax
import jax.numpy as jnp
from jax import lax
from jax.experimental import pallas as pl
from jax.experimental.pallas import tpu as pltpu
from jax.experimental.pallas import tpu_sc as plsc

N_NODES = 50000
N_EDGES = 800000
SEQ_LEN = 20
IN_FEAT = 5
HIDDEN = 128
GCN_HID = 16

NPAD = 50048
ROWS_PER_SUB = NPAD // 16
CH = 125
NCHUNK = N_EDGES // CH
NW = 32
CPW = NCHUNK // NW
KD = 8
NGROUP = CPW // KD


def _lstm_body(x_ref, wih_ref, whh_ref, b_ref, w1_ref, out_ref):
    x = x_ref[...]
    nb = x.shape[0]
    h = jnp.zeros((nb, HIDDEN), jnp.float32)
    c = jnp.zeros((nb, HIDDEN), jnp.float32)
    bf = jnp.bfloat16
    wih = wih_ref[...].astype(bf)
    whh = whh_ref[...].astype(bf)
    b = b_ref[...]
    dn = (((1,), (0,)), ((), ()))
    for t in range(SEQ_LEN):
        x_t = x[:, IN_FEAT * t:IN_FEAT * (t + 1)]
        gates = lax.dot_general(x_t.astype(bf), wih, dn,
                                preferred_element_type=jnp.float32)
        gates = gates + lax.dot_general(h.astype(bf), whh, dn,
                                        preferred_element_type=jnp.float32)
        gates = gates + b
        i = jax.nn.sigmoid(gates[:, 0:HIDDEN])
        f = jax.nn.sigmoid(gates[:, HIDDEN:2 * HIDDEN])
        g = jnp.tanh(gates[:, 2 * HIDDEN:3 * HIDDEN])
        o = jax.nn.sigmoid(gates[:, 3 * HIDDEN:4 * HIDDEN])
        c = f * c + i * g
        h = o * jnp.tanh(c)
    out_ref[...] = lax.dot_general(h, w1_ref[...], dn,
                                   preferred_element_type=jnp.float32)


def _lstm_xw1(x2d, wih_t, whh_t, bias, w1):
    nb = 5000
    grid = N_NODES // nb
    return pl.pallas_call(
        _lstm_body,
        grid=(grid,),
        in_specs=[
            pl.BlockSpec((nb, SEQ_LEN * IN_FEAT), lambda i: (i, 0)),
            pl.BlockSpec((IN_FEAT, 4 * HIDDEN), lambda i: (0, 0)),
            pl.BlockSpec((HIDDEN, 4 * HIDDEN), lambda i: (0, 0)),
            pl.BlockSpec((1, 4 * HIDDEN), lambda i: (0, 0)),
            pl.BlockSpec((HIDDEN, GCN_HID), lambda i: (0, 0)),
        ],
        out_specs=pl.BlockSpec((nb, GCN_HID), lambda i: (i, 0)),
        out_shape=jax.ShapeDtypeStruct((N_NODES, GCN_HID), jnp.float32),
    )(x2d, wih_t, whh_t, bias, w1)


def _sc_mesh():
    return plsc.VectorSubcoreMesh(core_axis_name="c", subcore_axis_name="s",
                                  num_cores=2, num_subcores=16)


@functools.partial(
    pl.kernel,
    out_type=jax.ShapeDtypeStruct((2, NPAD, 1), jnp.float32),
    mesh=_sc_mesh(),
    scratch_types=[
        pltpu.VMEM((CPW, CH), jnp.int32),
        pltpu.VMEM((CH, 1), jnp.float32),
        pltpu.VMEM_SHARED((NPAD, 1), jnp.float32),
        pltpu.SemaphoreType.DMA,
        pltpu.SemaphoreType.DMA,
    ],
    compiler_params=pltpu.CompilerParams(use_tc_tiling_on_sc=False),
)
def _sc_degree(dst_hbm, ones_hbm, zeros_hbm, out_hbm, didx, onev, acc,
               isem, ssem):
    cid = lax.axis_index("c")
    sid = lax.axis_index("s")
    wid = sid * 2 + cid
    base_rows = sid * ROWS_PER_SUB
    pltpu.async_copy(dst_hbm.at[pl.ds(wid * CPW, CPW)], didx, isem)
    pltpu.sync_copy(ones_hbm, onev)
    pltpu.sync_copy(zeros_hbm, acc.at[pl.ds(base_rows, ROWS_PER_SUB)])
    pltpu.make_async_copy(dst_hbm.at[pl.ds(wid * CPW, CPW)], didx, isem).wait()
    plsc.subcore_barrier()

    def group(gi, _):
        for b in range(KD):
            k = gi * KD + b
            pltpu.async_copy(onev, acc.at[didx.at[k]], ssem, add=True)
        for b in range(KD):
            k = gi * KD + b
            pltpu.make_async_copy(onev, acc.at[didx.at[k]], ssem).wait()
        return 0

    lax.fori_loop(0, NGROUP, group, 0)
    plsc.subcore_barrier()
    pltpu.sync_copy(acc.at[pl.ds(base_rows, ROWS_PER_SUB)],
                    out_hbm.at[cid, pl.ds(base_rows, ROWS_PER_SUB)])


@functools.partial(
    pl.kernel,
    out_type=jax.ShapeDtypeStruct((2, NPAD, GCN_HID), jnp.float32),
    mesh=_sc_mesh(),
    scratch_types=[
        pltpu.VMEM((CPW, CH), jnp.int32),
        pltpu.VMEM((CPW, CH), jnp.int32),
        pltpu.VMEM((KD, CH, GCN_HID), jnp.float32),
        pltpu.VMEM_SHARED((NPAD, GCN_HID), jnp.float32),
        pltpu.SemaphoreType.DMA,
        pltpu.SemaphoreType.DMA,
        pltpu.SemaphoreType.DMA,
    ],
    compiler_params=pltpu.CompilerParams(use_tc_tiling_on_sc=False),
)
def _sc_propagate(g_hbm, src_hbm, dst_hbm, zeros_hbm, out_hbm,
                  sidx, didx, rows, acc, isem, gsem, ssem):
    cid = lax.axis_index("c")
    sid = lax.axis_index("s")
    wid = sid * 2 + cid
    base_rows = sid * ROWS_PER_SUB
    pltpu.async_copy(src_hbm.at[pl.ds(wid * CPW, CPW)], sidx, isem)
    pltpu.async_copy(dst_hbm.at[pl.ds(wid * CPW, CPW)], didx, isem)
    pltpu.sync_copy(zeros_hbm, acc.at[pl.ds(base_rows, ROWS_PER_SUB)])
    pltpu.make_async_copy(src_hbm.at[pl.ds(wid * CPW, CPW)], sidx, isem).wait()
    pltpu.make_async_copy(dst_hbm.at[pl.ds(wid * CPW, CPW)], didx, isem).wait()
    plsc.subcore_barrier()

    def group(gi, _):
        for b in range(KD):
            k = gi * KD + b
            pltpu.async_copy(g_hbm.at[sidx.at[k]], rows.at[b], gsem)
        for b in range(KD):
            k = gi * KD + b
            pltpu.make_async_copy(g_hbm.at[sidx.at[k]], rows.at[b],
                                  gsem).wait()
            pltpu.async_copy(rows.at[b], acc.at[didx.at[k]], ssem, add=True)
        for b in range(KD):
            k = gi * KD + b
            pltpu.make_async_copy(rows.at[b], acc.at[didx.at[k]], ssem).wait()
        return 0

    lax.fori_loop(0, NGROUP, group, 0)
    plsc.subcore_barrier()
    pltpu.sync_copy(acc.at[pl.ds(base_rows, ROWS_PER_SUB)],
                    out_hbm.at[cid, pl.ds(base_rows, ROWS_PER_SUB)])


def _scale1_body(degp_ref, xw1_ref, dinv_ref, g1_ref):
    deg = degp_ref[0] + degp_ref[1] + 1.0
    dinv = lax.rsqrt(deg)
    dinv_ref[...] = dinv
    g1_ref[...] = dinv * xw1_ref[...]


def _scale2_body(accp_ref, xw1_ref, dinv_ref, b1_ref, y_ref, g2_ref):
    dinv = dinv_ref[...]
    acc = accp_ref[0] + accp_ref[1]
    p1 = dinv * acc + (dinv * dinv) * xw1_ref[...]
    y = jnp.maximum(p1 + b1_ref[...], 0.0)
    y_ref[...] = y
    g2_ref[...] = dinv * y


def _final_body(accp_ref, y_ref, dinv_ref, w2_ref, b2_ref, wfc_ref, bfc_ref,
                out_ref):
    dinv = dinv_ref[...]
    acc = accp_ref[0] + accp_ref[1]
    p2 = dinv * acc + (dinv * dinv) * y_ref[...]
    dn_t = (((1,), (1,)), ((), ()))
    v = lax.dot_general(w2_ref[...], wfc_ref[...], dn_t,
                        preferred_element_type=jnp.float32)
    c = lax.dot_general(b2_ref[...], wfc_ref[...], dn_t,
                        preferred_element_type=jnp.float32)
    dn = (((1,), (0,)), ((), ()))
    z = lax.dot_general(p2, v, dn, preferred_element_type=jnp.float32)
    z = z + c + bfc_ref[...]
    out_ref[...] = jnp.where(z > 0, z, 0.2 * z)


_NB2 = 5000
_GRID2 = N_NODES // _NB2


def _scale1(degp, xw1):
    return pl.pallas_call(
        _scale1_body,
        grid=(_GRID2,),
        in_specs=[
            pl.BlockSpec((2, _NB2, 1), lambda i: (0, i, 0)),
            pl.BlockSpec((_NB2, GCN_HID), lambda i: (i, 0)),
        ],
        out_specs=[
            pl.BlockSpec((_NB2, 1), lambda i: (i, 0)),
            pl.BlockSpec((_NB2, GCN_HID), lambda i: (i, 0)),
        ],
        out_shape=[
            jax.ShapeDtypeStruct((N_NODES, 1), jnp.float32),
            jax.ShapeDtypeStruct((N_NODES, GCN_HID), jnp.float32),
        ],
    )(degp, xw1)


def _scale2(accp, xw1, dinv, b1):
    return pl.pallas_call(
        _scale2_body,
        grid=(_GRID2,),
        in_specs=[
            pl.BlockSpec((2, _NB2, GCN_HID), lambda i: (0, i, 0)),
            pl.BlockSpec((_NB2, GCN_HID), lambda i: (i, 0)),
            pl.BlockSpec((_NB2, 1), lambda i: (i, 0)),
            pl.BlockSpec((1, GCN_HID), lambda i: (0, 0)),
        ],
        out_specs=[
            pl.BlockSpec((_NB2, GCN_HID), lambda i: (i, 0)),
            pl.BlockSpec((_NB2, GCN_HID), lambda i: (i, 0)),
        ],
        out_shape=[
            jax.ShapeDtypeStruct((N_NODES, GCN_HID), jnp.float32),
            jax.ShapeDtypeStruct((N_NODES, GCN_HID), jnp.float32),
        ],
    )(accp, xw1, dinv, b1)


def _final(accp, y, dinv, w2, b2, wfc, bfc):
    return pl.pallas_call(
        _final_body,
        grid=(_GRID2,),
        in_specs=[
            pl.BlockSpec((2, _NB2, GCN_HID), lambda i: (0, i, 0)),
            pl.BlockSpec((_NB2, GCN_HID), lambda i: (i, 0)),
            pl.BlockSpec((_NB2, 1), lambda i: (i, 0)),
            pl.BlockSpec((GCN_HID, HIDDEN), lambda i: (0, 0)),
            pl.BlockSpec((1, HIDDEN), lambda i: (0, 0)),
            pl.BlockSpec((1, HIDDEN), lambda i: (0, 0)),
            pl.BlockSpec((1, 1), lambda i: (0, 0)),
        ],
        out_specs=pl.BlockSpec((_NB2, 1), lambda i: (i, 0)),
        out_shape=jax.ShapeDtypeStruct((N_NODES, 1), jnp.float32),
    )(accp, y, dinv, w2, b2, wfc, bfc)


def kernel(inputs, edge_index, W_ih, W_hh, b_ih, b_hh, W1, b1, W2, b2,
           W_fc, b_fc):
    src = edge_index[0].astype(jnp.int32).reshape(NCHUNK, CH)
    dst = edge_index[1].astype(jnp.int32).reshape(NCHUNK, CH)
    x2d = inputs.reshape(N_NODES, SEQ_LEN * IN_FEAT)
    wih_t = W_ih.T
    whh_t = W_hh.T
    bias = (b_ih + b_hh).reshape(1, 4 * HIDDEN)

    xw1 = _lstm_xw1(x2d, wih_t, whh_t, bias, W1)

    ones_e = jnp.ones((CH, 1), jnp.float32)
    zeros1 = jnp.zeros((ROWS_PER_SUB, 1), jnp.float32)
    zeros16 = jnp.zeros((ROWS_PER_SUB, GCN_HID), jnp.float32)

    degp = _sc_degree(dst, ones_e, zeros1)
    degp = degp[:, :N_NODES]
    dinv, g1 = _scale1(degp, xw1)

    acc1 = _sc_propagate(g1, src, dst, zeros16)
    y, g2 = _scale2(acc1[:, :N_NODES], xw1, dinv, b1.reshape(1, GCN_HID))

    acc2 = _sc_propagate(g2, src, dst, zeros16)
    pred = _final(acc2[:, :N_NODES], y, dinv, W2,
                  b2.reshape(1, HIDDEN), W_fc, b_fc.reshape(1, 1))
    return pred


# speedup vs baseline: 1.1917x; 1.0036x over previous
import functools

import jax
import jax.numpy as jnp
from jax import lax
from jax.experimental import pallas as pl
from jax.experimental.pallas import tpu as pltpu
from jax.experimental.pallas import tpu_sc as plsc

N_NODES = 50000
N_EDGES = 800000
SEQ_LEN = 20
IN_FEAT = 5
HIDDEN = 128
GCN_HID = 16

NPAD = 50048
ROWS_PER_SUB = NPAD // 16
CH = 125
NCHUNK = N_EDGES // CH
NW = 32
CPW = NCHUNK // NW
KD = 8
NGROUP = CPW // KD


def _lstm_body(x_ref, wih_ref, whh_ref, b_ref, w1_ref, out_ref):
    x = x_ref[...]
    nb = x.shape[0]
    h = jnp.zeros((nb, HIDDEN), jnp.float32)
    c = jnp.zeros((nb, HIDDEN), jnp.float32)
    bf = jnp.bfloat16
    wih = wih_ref[...].astype(bf)
    whh = whh_ref[...].astype(bf)
    b = b_ref[...]
    dn = (((1,), (0,)), ((), ()))
    for t in range(SEQ_LEN):
        x_t = x[:, IN_FEAT * t:IN_FEAT * (t + 1)]
        gates = lax.dot_general(x_t.astype(bf), wih, dn,
                                preferred_element_type=jnp.float32)
        gates = gates + lax.dot_general(h.astype(bf), whh, dn,
                                        preferred_element_type=jnp.float32)
        gates = gates + b
        i = jax.nn.sigmoid(gates[:, 0:HIDDEN])
        f = jax.nn.sigmoid(gates[:, HIDDEN:2 * HIDDEN])
        g = jnp.tanh(gates[:, 2 * HIDDEN:3 * HIDDEN])
        o = jax.nn.sigmoid(gates[:, 3 * HIDDEN:4 * HIDDEN])
        c = f * c + i * g
        h = o * jnp.tanh(c)
    out_ref[...] = lax.dot_general(h, w1_ref[...], dn,
                                   preferred_element_type=jnp.float32)


def _lstm_xw1(x2d, wih_t, whh_t, bias, w1):
    nb = 1000
    grid = N_NODES // nb
    return pl.pallas_call(
        _lstm_body,
        grid=(grid,),
        in_specs=[
            pl.BlockSpec((nb, SEQ_LEN * IN_FEAT), lambda i: (i, 0)),
            pl.BlockSpec((IN_FEAT, 4 * HIDDEN), lambda i: (0, 0)),
            pl.BlockSpec((HIDDEN, 4 * HIDDEN), lambda i: (0, 0)),
            pl.BlockSpec((1, 4 * HIDDEN), lambda i: (0, 0)),
            pl.BlockSpec((HIDDEN, GCN_HID), lambda i: (0, 0)),
        ],
        out_specs=pl.BlockSpec((nb, GCN_HID), lambda i: (i, 0)),
        out_shape=jax.ShapeDtypeStruct((N_NODES, GCN_HID), jnp.float32),
    )(x2d, wih_t, whh_t, bias, w1)


def _sc_mesh():
    return plsc.VectorSubcoreMesh(core_axis_name="c", subcore_axis_name="s",
                                  num_cores=2, num_subcores=16)


@functools.partial(
    pl.kernel,
    out_type=jax.ShapeDtypeStruct((2, NPAD, 1), jnp.float32),
    mesh=_sc_mesh(),
    scratch_types=[
        pltpu.VMEM((CPW, CH), jnp.int32),
        pltpu.VMEM((CH, 1), jnp.float32),
        pltpu.VMEM_SHARED((NPAD, 1), jnp.float32),
        pltpu.SemaphoreType.DMA,
        pltpu.SemaphoreType.DMA,
    ],
    compiler_params=pltpu.CompilerParams(use_tc_tiling_on_sc=False),
)
def _sc_degree(dst_hbm, ones_hbm, zeros_hbm, out_hbm, didx, onev, acc,
               isem, ssem):
    cid = lax.axis_index("c")
    sid = lax.axis_index("s")
    wid = sid * 2 + cid
    base_rows = sid * ROWS_PER_SUB
    pltpu.async_copy(dst_hbm.at[pl.ds(wid * CPW, CPW)], didx, isem)
    pltpu.sync_copy(ones_hbm, onev)
    pltpu.sync_copy(zeros_hbm, acc.at[pl.ds(base_rows, ROWS_PER_SUB)])
    pltpu.make_async_copy(dst_hbm.at[pl.ds(wid * CPW, CPW)], didx, isem).wait()
    plsc.subcore_barrier()

    def group(gi, _):
        for b in range(KD):
            k = gi * KD + b
            pltpu.async_copy(onev, acc.at[didx.at[k]], ssem, add=True)
        for b in range(KD):
            k = gi * KD + b
            pltpu.make_async_copy(onev, acc.at[didx.at[k]], ssem).wait()
        return 0

    lax.fori_loop(0, NGROUP, group, 0)
    plsc.subcore_barrier()
    pltpu.sync_copy(acc.at[pl.ds(base_rows, ROWS_PER_SUB)],
                    out_hbm.at[cid, pl.ds(base_rows, ROWS_PER_SUB)])


@functools.partial(
    pl.kernel,
    out_type=jax.ShapeDtypeStruct((2, NPAD, GCN_HID), jnp.float32),
    mesh=_sc_mesh(),
    scratch_types=[
        pltpu.VMEM((CPW, CH), jnp.int32),
        pltpu.VMEM((CPW, CH), jnp.int32),
        pltpu.VMEM((KD, CH, GCN_HID), jnp.float32),
        pltpu.VMEM_SHARED((NPAD, GCN_HID), jnp.float32),
        pltpu.SemaphoreType.DMA,
        pltpu.SemaphoreType.DMA,
        pltpu.SemaphoreType.DMA,
    ],
    compiler_params=pltpu.CompilerParams(use_tc_tiling_on_sc=False),
)
def _sc_propagate(g_hbm, src_hbm, dst_hbm, zeros_hbm, out_hbm,
                  sidx, didx, rows, acc, isem, gsem, ssem):
    cid = lax.axis_index("c")
    sid = lax.axis_index("s")
    wid = sid * 2 + cid
    base_rows = sid * ROWS_PER_SUB
    pltpu.async_copy(src_hbm.at[pl.ds(wid * CPW, CPW)], sidx, isem)
    pltpu.async_copy(dst_hbm.at[pl.ds(wid * CPW, CPW)], didx, isem)
    pltpu.sync_copy(zeros_hbm, acc.at[pl.ds(base_rows, ROWS_PER_SUB)])
    pltpu.make_async_copy(src_hbm.at[pl.ds(wid * CPW, CPW)], sidx, isem).wait()
    pltpu.make_async_copy(dst_hbm.at[pl.ds(wid * CPW, CPW)], didx, isem).wait()
    plsc.subcore_barrier()

    def group(gi, _):
        for b in range(KD):
            k = gi * KD + b
            pltpu.async_copy(g_hbm.at[sidx.at[k]], rows.at[b], gsem)
        for b in range(KD):
            k = gi * KD + b
            pltpu.make_async_copy(g_hbm.at[sidx.at[k]], rows.at[b],
                                  gsem).wait()
            pltpu.async_copy(rows.at[b], acc.at[didx.at[k]], ssem, add=True)
        for b in range(KD):
            k = gi * KD + b
            pltpu.make_async_copy(rows.at[b], acc.at[didx.at[k]], ssem).wait()
        return 0

    lax.fori_loop(0, NGROUP, group, 0)
    plsc.subcore_barrier()
    pltpu.sync_copy(acc.at[pl.ds(base_rows, ROWS_PER_SUB)],
                    out_hbm.at[cid, pl.ds(base_rows, ROWS_PER_SUB)])


def _scale1_body(degp_ref, xw1_ref, dinv_ref, g1_ref):
    deg = degp_ref[0] + degp_ref[1] + 1.0
    dinv = lax.rsqrt(deg)
    dinv_ref[...] = dinv
    g1_ref[...] = dinv * xw1_ref[...]


def _scale2_body(accp_ref, xw1_ref, dinv_ref, b1_ref, y_ref, g2_ref):
    dinv = dinv_ref[...]
    acc = accp_ref[0] + accp_ref[1]
    p1 = dinv * acc + (dinv * dinv) * xw1_ref[...]
    y = jnp.maximum(p1 + b1_ref[...], 0.0)
    y_ref[...] = y
    g2_ref[...] = dinv * y


def _final_body(accp_ref, y_ref, dinv_ref, w2_ref, b2_ref, wfc_ref, bfc_ref,
                out_ref):
    dinv = dinv_ref[...]
    acc = accp_ref[0] + accp_ref[1]
    p2 = dinv * acc + (dinv * dinv) * y_ref[...]
    dn_t = (((1,), (1,)), ((), ()))
    v = lax.dot_general(w2_ref[...], wfc_ref[...], dn_t,
                        preferred_element_type=jnp.float32)
    c = lax.dot_general(b2_ref[...], wfc_ref[...], dn_t,
                        preferred_element_type=jnp.float32)
    dn = (((1,), (0,)), ((), ()))
    z = lax.dot_general(p2, v, dn, preferred_element_type=jnp.float32)
    z = z + c + bfc_ref[...]
    out_ref[...] = jnp.where(z > 0, z, 0.2 * z)


_NB2 = 5000
_GRID2 = N_NODES // _NB2


def _scale1(degp, xw1):
    return pl.pallas_call(
        _scale1_body,
        grid=(_GRID2,),
        in_specs=[
            pl.BlockSpec((2, _NB2, 1), lambda i: (0, i, 0)),
            pl.BlockSpec((_NB2, GCN_HID), lambda i: (i, 0)),
        ],
        out_specs=[
            pl.BlockSpec((_NB2, 1), lambda i: (i, 0)),
            pl.BlockSpec((_NB2, GCN_HID), lambda i: (i, 0)),
        ],
        out_shape=[
            jax.ShapeDtypeStruct((N_NODES, 1), jnp.float32),
            jax.ShapeDtypeStruct((N_NODES, GCN_HID), jnp.float32),
        ],
    )(degp, xw1)


def _scale2(accp, xw1, dinv, b1):
    return pl.pallas_call(
        _scale2_body,
        grid=(_GRID2,),
        in_specs=[
            pl.BlockSpec((2, _NB2, GCN_HID), lambda i: (0, i, 0)),
            pl.BlockSpec((_NB2, GCN_HID), lambda i: (i, 0)),
            pl.BlockSpec((_NB2, 1), lambda i: (i, 0)),
            pl.BlockSpec((1, GCN_HID), lambda i: (0, 0)),
        ],
        out_specs=[
            pl.BlockSpec((_NB2, GCN_HID), lambda i: (i, 0)),
            pl.BlockSpec((_NB2, GCN_HID), lambda i: (i, 0)),
        ],
        out_shape=[
            jax.ShapeDtypeStruct((N_NODES, GCN_HID), jnp.float32),
            jax.ShapeDtypeStruct((N_NODES, GCN_HID), jnp.float32),
        ],
    )(accp, xw1, dinv, b1)


def _final(accp, y, dinv, w2, b2, wfc, bfc):
    return pl.pallas_call(
        _final_body,
        grid=(_GRID2,),
        in_specs=[
            pl.BlockSpec((2, _NB2, GCN_HID), lambda i: (0, i, 0)),
            pl.BlockSpec((_NB2, GCN_HID), lambda i: (i, 0)),
            pl.BlockSpec((_NB2, 1), lambda i: (i, 0)),
            pl.BlockSpec((GCN_HID, HIDDEN), lambda i: (0, 0)),
            pl.BlockSpec((1, HIDDEN), lambda i: (0, 0)),
            pl.BlockSpec((1, HIDDEN), lambda i: (0, 0)),
            pl.BlockSpec((1, 1), lambda i: (0, 0)),
        ],
        out_specs=pl.BlockSpec((_NB2, 1), lambda i: (i, 0)),
        out_shape=jax.ShapeDtypeStruct((N_NODES, 1), jnp.float32),
    )(accp, y, dinv, w2, b2, wfc, bfc)


def kernel(inputs, edge_index, W_ih, W_hh, b_ih, b_hh, W1, b1, W2, b2,
           W_fc, b_fc):
    src = edge_index[0].astype(jnp.int32).reshape(NCHUNK, CH)
    dst = edge_index[1].astype(jnp.int32).reshape(NCHUNK, CH)
    x2d = inputs.reshape(N_NODES, SEQ_LEN * IN_FEAT)
    wih_t = W_ih.T
    whh_t = W_hh.T
    bias = (b_ih + b_hh).reshape(1, 4 * HIDDEN)

    xw1 = _lstm_xw1(x2d, wih_t, whh_t, bias, W1)

    ones_e = jnp.ones((CH, 1), jnp.float32)
    zeros1 = jnp.zeros((ROWS_PER_SUB, 1), jnp.float32)
    zeros16 = jnp.zeros((ROWS_PER_SUB, GCN_HID), jnp.float32)

    degp = _sc_degree(dst, ones_e, zeros1)
    degp = degp[:, :N_NODES]
    dinv, g1 = _scale1(degp, xw1)

    acc1 = _sc_propagate(g1, src, dst, zeros16)
    y, g2 = _scale2(acc1[:, :N_NODES], xw1, dinv, b1.reshape(1, GCN_HID))

    acc2 = _sc_propagate(g2, src, dst, zeros16)
    pred = _final(acc2[:, :N_NODES], y, dinv, W2,
                  b2.reshape(1, HIDDEN), W_fc, b_fc.reshape(1, 1))
    return pred


# speedup vs baseline: 1.2638x; 1.0605x over previous
import functools

import jax
import jax.numpy as jnp
from jax import lax
from jax.experimental import pallas as pl
from jax.experimental.pallas import tpu as pltpu
from jax.experimental.pallas import tpu_sc as plsc

N_NODES = 50000
N_EDGES = 800000
SEQ_LEN = 20
IN_FEAT = 5
HIDDEN = 128
GCN_HID = 16

NPAD = 50048
ROWS_PER_SUB = NPAD // 16
CH = 125
NCHUNK = N_EDGES // CH
NW = 32
CPW = NCHUNK // NW
KD = 8
NGROUP = CPW // KD


def _lstm_body(x_ref, wih_ref, whh_ref, b_ref, w1_ref, degp_ref,
               xw1_ref, dinv_ref, g1_ref):
    x = x_ref[...]
    nb = x.shape[0]
    h = jnp.zeros((nb, HIDDEN), jnp.float32)
    c = jnp.zeros((nb, HIDDEN), jnp.float32)
    bf = jnp.bfloat16
    wih = wih_ref[...].astype(bf)
    whh = whh_ref[...].astype(bf)
    b = b_ref[...]
    dn = (((1,), (0,)), ((), ()))
    for t in range(SEQ_LEN):
        x_t = x[:, IN_FEAT * t:IN_FEAT * (t + 1)]
        gates = lax.dot_general(x_t.astype(bf), wih, dn,
                                preferred_element_type=jnp.float32)
        gates = gates + lax.dot_general(h.astype(bf), whh, dn,
                                        preferred_element_type=jnp.float32)
        gates = gates + b
        i = jax.nn.sigmoid(gates[:, 0:HIDDEN])
        f = jax.nn.sigmoid(gates[:, HIDDEN:2 * HIDDEN])
        g = jnp.tanh(gates[:, 2 * HIDDEN:3 * HIDDEN])
        o = jax.nn.sigmoid(gates[:, 3 * HIDDEN:4 * HIDDEN])
        c = f * c + i * g
        h = o * jnp.tanh(c)
    xw1 = lax.dot_general(h, w1_ref[...], dn,
                          preferred_element_type=jnp.float32)
    deg = degp_ref[0] + degp_ref[1] + 1.0
    dinv = lax.rsqrt(deg)
    xw1_ref[...] = xw1
    dinv_ref[...] = dinv
    g1_ref[...] = dinv * xw1


def _lstm_xw1(x2d, wih_t, whh_t, bias, w1, degp):
    nb = 2000
    grid = N_NODES // nb
    return pl.pallas_call(
        _lstm_body,
        grid=(grid,),
        in_specs=[
            pl.BlockSpec((nb, SEQ_LEN * IN_FEAT), lambda i: (i, 0)),
            pl.BlockSpec((IN_FEAT, 4 * HIDDEN), lambda i: (0, 0)),
            pl.BlockSpec((HIDDEN, 4 * HIDDEN), lambda i: (0, 0)),
            pl.BlockSpec((1, 4 * HIDDEN), lambda i: (0, 0)),
            pl.BlockSpec((HIDDEN, GCN_HID), lambda i: (0, 0)),
            pl.BlockSpec((2, nb, 1), lambda i: (0, i, 0)),
        ],
        out_specs=[
            pl.BlockSpec((nb, GCN_HID), lambda i: (i, 0)),
            pl.BlockSpec((nb, 1), lambda i: (i, 0)),
            pl.BlockSpec((nb, GCN_HID), lambda i: (i, 0)),
        ],
        out_shape=[
            jax.ShapeDtypeStruct((N_NODES, GCN_HID), jnp.float32),
            jax.ShapeDtypeStruct((N_NODES, 1), jnp.float32),
            jax.ShapeDtypeStruct((N_NODES, GCN_HID), jnp.float32),
        ],
    )(x2d, wih_t, whh_t, bias, w1, degp)


def _sc_mesh():
    return plsc.VectorSubcoreMesh(core_axis_name="c", subcore_axis_name="s",
                                  num_cores=2, num_subcores=16)


@functools.partial(
    pl.kernel,
    out_type=jax.ShapeDtypeStruct((2, NPAD, 1), jnp.float32),
    mesh=_sc_mesh(),
    scratch_types=[
        pltpu.VMEM((CPW, CH), jnp.int32),
        pltpu.VMEM((CH, 1), jnp.float32),
        pltpu.VMEM_SHARED((NPAD, 1), jnp.float32),
        pltpu.SemaphoreType.DMA,
        pltpu.SemaphoreType.DMA,
    ],
    compiler_params=pltpu.CompilerParams(use_tc_tiling_on_sc=False),
)
def _sc_degree(dst_hbm, ones_hbm, zeros_hbm, out_hbm, didx, onev, acc,
               isem, ssem):
    cid = lax.axis_index("c")
    sid = lax.axis_index("s")
    wid = sid * 2 + cid
    base_rows = sid * ROWS_PER_SUB
    pltpu.async_copy(dst_hbm.at[pl.ds(wid * CPW, CPW)], didx, isem)
    pltpu.sync_copy(ones_hbm, onev)
    pltpu.sync_copy(zeros_hbm, acc.at[pl.ds(base_rows, ROWS_PER_SUB)])
    pltpu.make_async_copy(dst_hbm.at[pl.ds(wid * CPW, CPW)], didx, isem).wait()
    plsc.subcore_barrier()

    def group(gi, _):
        for b in range(KD):
            k = gi * KD + b
            pltpu.async_copy(onev, acc.at[didx.at[k]], ssem, add=True)
        for b in range(KD):
            k = gi * KD + b
            pltpu.make_async_copy(onev, acc.at[didx.at[k]], ssem).wait()
        return 0

    lax.fori_loop(0, NGROUP, group, 0)
    plsc.subcore_barrier()
    pltpu.sync_copy(acc.at[pl.ds(base_rows, ROWS_PER_SUB)],
                    out_hbm.at[cid, pl.ds(base_rows, ROWS_PER_SUB)])


@functools.partial(
    pl.kernel,
    out_type=jax.ShapeDtypeStruct((2, NPAD, GCN_HID), jnp.float32),
    mesh=_sc_mesh(),
    scratch_types=[
        pltpu.VMEM((CPW, CH), jnp.int32),
        pltpu.VMEM((CPW, CH), jnp.int32),
        pltpu.VMEM((KD, CH, GCN_HID), jnp.float32),
        pltpu.VMEM_SHARED((NPAD, GCN_HID), jnp.float32),
        pltpu.SemaphoreType.DMA,
        pltpu.SemaphoreType.DMA,
        pltpu.SemaphoreType.DMA,
    ],
    compiler_params=pltpu.CompilerParams(use_tc_tiling_on_sc=False),
)
def _sc_propagate(g_hbm, src_hbm, dst_hbm, zeros_hbm, out_hbm,
                  sidx, didx, rows, acc, isem, gsem, ssem):
    cid = lax.axis_index("c")
    sid = lax.axis_index("s")
    wid = sid * 2 + cid
    base_rows = sid * ROWS_PER_SUB
    pltpu.async_copy(src_hbm.at[pl.ds(wid * CPW, CPW)], sidx, isem)
    pltpu.async_copy(dst_hbm.at[pl.ds(wid * CPW, CPW)], didx, isem)
    pltpu.sync_copy(zeros_hbm, acc.at[pl.ds(base_rows, ROWS_PER_SUB)])
    pltpu.make_async_copy(src_hbm.at[pl.ds(wid * CPW, CPW)], sidx, isem).wait()
    pltpu.make_async_copy(dst_hbm.at[pl.ds(wid * CPW, CPW)], didx, isem).wait()
    plsc.subcore_barrier()

    def group(gi, _):
        for b in range(KD):
            k = gi * KD + b
            pltpu.async_copy(g_hbm.at[sidx.at[k]], rows.at[b], gsem)
        for b in range(KD):
            k = gi * KD + b
            pltpu.make_async_copy(g_hbm.at[sidx.at[k]], rows.at[b],
                                  gsem).wait()
            pltpu.async_copy(rows.at[b], acc.at[didx.at[k]], ssem, add=True)
        for b in range(KD):
            k = gi * KD + b
            pltpu.make_async_copy(rows.at[b], acc.at[didx.at[k]], ssem).wait()
        return 0

    lax.fori_loop(0, NGROUP, group, 0)
    plsc.subcore_barrier()
    pltpu.sync_copy(acc.at[pl.ds(base_rows, ROWS_PER_SUB)],
                    out_hbm.at[cid, pl.ds(base_rows, ROWS_PER_SUB)])


def _scale1_body(degp_ref, xw1_ref, dinv_ref, g1_ref):
    deg = degp_ref[0] + degp_ref[1] + 1.0
    dinv = lax.rsqrt(deg)
    dinv_ref[...] = dinv
    g1_ref[...] = dinv * xw1_ref[...]


def _scale2_body(accp_ref, xw1_ref, dinv_ref, b1_ref, y_ref, g2_ref):
    dinv = dinv_ref[...]
    acc = accp_ref[0] + accp_ref[1]
    p1 = dinv * acc + (dinv * dinv) * xw1_ref[...]
    y = jnp.maximum(p1 + b1_ref[...], 0.0)
    y_ref[...] = y
    g2_ref[...] = dinv * y


def _final_body(accp_ref, y_ref, dinv_ref, w2_ref, b2_ref, wfc_ref, bfc_ref,
                out_ref):
    dinv = dinv_ref[...]
    acc = accp_ref[0] + accp_ref[1]
    p2 = dinv * acc + (dinv * dinv) * y_ref[...]
    dn_t = (((1,), (1,)), ((), ()))
    v = lax.dot_general(w2_ref[...], wfc_ref[...], dn_t,
                        preferred_element_type=jnp.float32)
    c = lax.dot_general(b2_ref[...], wfc_ref[...], dn_t,
                        preferred_element_type=jnp.float32)
    dn = (((1,), (0,)), ((), ()))
    z = lax.dot_general(p2, v, dn, preferred_element_type=jnp.float32)
    z = z + c + bfc_ref[...]
    out_ref[...] = jnp.where(z > 0, z, 0.2 * z)


_NB2 = 5000
_GRID2 = N_NODES // _NB2


def _scale1(degp, xw1):
    return pl.pallas_call(
        _scale1_body,
        grid=(_GRID2,),
        in_specs=[
            pl.BlockSpec((2, _NB2, 1), lambda i: (0, i, 0)),
            pl.BlockSpec((_NB2, GCN_HID), lambda i: (i, 0)),
        ],
        out_specs=[
            pl.BlockSpec((_NB2, 1), lambda i: (i, 0)),
            pl.BlockSpec((_NB2, GCN_HID), lambda i: (i, 0)),
        ],
        out_shape=[
            jax.ShapeDtypeStruct((N_NODES, 1), jnp.float32),
            jax.ShapeDtypeStruct((N_NODES, GCN_HID), jnp.float32),
        ],
    )(degp, xw1)


def _scale2(accp, xw1, dinv, b1):
    return pl.pallas_call(
        _scale2_body,
        grid=(_GRID2,),
        in_specs=[
            pl.BlockSpec((2, _NB2, GCN_HID), lambda i: (0, i, 0)),
            pl.BlockSpec((_NB2, GCN_HID), lambda i: (i, 0)),
            pl.BlockSpec((_NB2, 1), lambda i: (i, 0)),
            pl.BlockSpec((1, GCN_HID), lambda i: (0, 0)),
        ],
        out_specs=[
            pl.BlockSpec((_NB2, GCN_HID), lambda i: (i, 0)),
            pl.BlockSpec((_NB2, GCN_HID), lambda i: (i, 0)),
        ],
        out_shape=[
            jax.ShapeDtypeStruct((N_NODES, GCN_HID), jnp.float32),
            jax.ShapeDtypeStruct((N_NODES, GCN_HID), jnp.float32),
        ],
    )(accp, xw1, dinv, b1)


def _final(accp, y, dinv, w2, b2, wfc, bfc):
    return pl.pallas_call(
        _final_body,
        grid=(_GRID2,),
        in_specs=[
            pl.BlockSpec((2, _NB2, GCN_HID), lambda i: (0, i, 0)),
            pl.BlockSpec((_NB2, GCN_HID), lambda i: (i, 0)),
            pl.BlockSpec((_NB2, 1), lambda i: (i, 0)),
            pl.BlockSpec((GCN_HID, HIDDEN), lambda i: (0, 0)),
            pl.BlockSpec((1, HIDDEN), lambda i: (0, 0)),
            pl.BlockSpec((1, HIDDEN), lambda i: (0, 0)),
            pl.BlockSpec((1, 1), lambda i: (0, 0)),
        ],
        out_specs=pl.BlockSpec((_NB2, 1), lambda i: (i, 0)),
        out_shape=jax.ShapeDtypeStruct((N_NODES, 1), jnp.float32),
    )(accp, y, dinv, w2, b2, wfc, bfc)


def kernel(inputs, edge_index, W_ih, W_hh, b_ih, b_hh, W1, b1, W2, b2,
           W_fc, b_fc):
    src = edge_index[0].astype(jnp.int32).reshape(NCHUNK, CH)
    dst = edge_index[1].astype(jnp.int32).reshape(NCHUNK, CH)
    x2d = inputs.reshape(N_NODES, SEQ_LEN * IN_FEAT)
    wih_t = W_ih.T
    whh_t = W_hh.T
    bias = (b_ih + b_hh).reshape(1, 4 * HIDDEN)

    ones_e = jnp.ones((CH, 1), jnp.float32)
    zeros1 = jnp.zeros((ROWS_PER_SUB, 1), jnp.float32)
    zeros16 = jnp.zeros((ROWS_PER_SUB, GCN_HID), jnp.float32)

    degp = _sc_degree(dst, ones_e, zeros1)
    degp = degp[:, :N_NODES]
    xw1, dinv, g1 = _lstm_xw1(x2d, wih_t, whh_t, bias, W1, degp)

    acc1 = _sc_propagate(g1, src, dst, zeros16)
    y, g2 = _scale2(acc1[:, :N_NODES], xw1, dinv, b1.reshape(1, GCN_HID))

    acc2 = _sc_propagate(g2, src, dst, zeros16)
    pred = _final(acc2[:, :N_NODES], y, dinv, W2,
                  b2.reshape(1, HIDDEN), W_fc, b_fc.reshape(1, 1))
    return pred
